# Initial kernel scaffold; baseline (speedup 1.0000x reference)
import jax
import jax.numpy as jnp
from jax.experimental import pallas as pl


def kernel(review_feat, ci, edge_index, W):
    raise NotImplementedError("write your pallas kernel here")


# speedup vs baseline: 1.5754x; 1.5754x over previous
import functools

import jax
import jax.numpy as jnp
from jax import lax
from jax.experimental import pallas as pl
from jax.experimental.pallas import tpu as pltpu
from jax.experimental.pallas import tpu_sc as plsc

N = 50000
E = 800000
D = 64

CHUNK = 80
NS = 16
NC = 2
NCHUNK = E // CHUNK
WINDOWS = NCHUNK // NS
HALF_D = D // NC
ROWS_PER_TILE = N // NS
ZROWS = 125
ZFLAT = ZROWS * HALF_D
WB_ROWS = 3128
WB_LAST = N - (NS - 1) * WB_ROWS
UPD = CHUNK * HALF_D


def _sc_body(review3d, ci_flat, src3, dst3, h_out,
             acc1d, rows, upd, idx, src_buf, dst_buf, civ_buf):
    c = lax.axis_index("c")
    s = lax.axis_index("s")
    col0 = c * HALF_D

    def zstore(i, carry):
        upd[pl.ds(i * 16, 16)] = jnp.zeros((16,), jnp.float32)
        return carry
    lax.fori_loop(0, UPD // 16, zstore, 0)
    NQ = (N * HALF_D) // UPD
    QPT = NQ // NS
    for k in range(QPT):
        pltpu.sync_copy(upd, acc1d.at[pl.ds((s * QPT + k) * UPD, UPD)])

    @pl.when(s == NS - 1)
    def _():
        pltpu.sync_copy(upd, acc1d.at[pl.ds((NQ - 1) * UPD, UPD)])

    plsc.subcore_barrier()

    iota16 = lax.iota(jnp.int32, 16)
    col0v = jnp.full((16,), 1, jnp.int32) * col0

    def window(w, carry):
        cb = s * WINDOWS + w
        pltpu.sync_copy(review3d.at[cb], rows)
        pltpu.sync_copy(src3.at[cb], src_buf)
        pltpu.sync_copy(dst3.at[cb], dst_buf)
        pltpu.sync_copy(ci_flat.at[src_buf.at[0]], civ_buf.at[0])

        for g in range(CHUNK // 16):
            sl = pl.ds(g * 16, 16)
            civ = civ_buf[0, sl]
            dstv = dst_buf[0, sl]
            dst32 = dstv * HALF_D
            eidx = g * 16 + iota16
            pos0 = eidx * HALF_D

            def dbody(d, carry2, civ=civ, dst32=dst32, eidx=eidx, pos0=pos0):
                dvec = jnp.full((16,), 1, jnp.int32) * d
                col = plsc.load_gather(rows, [eidx, col0v + dvec])
                plsc.store_scatter(upd, [pos0 + dvec], col * civ)
                plsc.store_scatter(idx, [pos0 + dvec], dst32 + dvec)
                return carry2
            lax.fori_loop(0, HALF_D, dbody, carry, unroll=4)

        pltpu.sync_copy(upd, acc1d.at[idx], add=True)
        return carry
    lax.fori_loop(0, WINDOWS, window, 0)

    plsc.subcore_barrier()

    hbase = c * (N * HALF_D)
    for k in range(QPT):
        off = (s * QPT + k) * UPD
        pltpu.sync_copy(acc1d.at[pl.ds(off, UPD)], upd)
        pltpu.sync_copy(upd, h_out.at[pl.ds(hbase + off, UPD)])

    @pl.when(s == NS - 1)
    def _():
        off = (NQ - 1) * UPD
        pltpu.sync_copy(acc1d.at[pl.ds(off, UPD)], upd)
        pltpu.sync_copy(upd, h_out.at[pl.ds(hbase + off, UPD)])


def _sc_aggregate(review3d, ci_flat, src3, dst3):
    mesh = plsc.VectorSubcoreMesh(core_axis_name="c", subcore_axis_name="s")
    return pl.kernel(
        _sc_body,
        out_type=jax.ShapeDtypeStruct((NC * N * HALF_D,), jnp.float32),
        mesh=mesh,
        compiler_params=pltpu.CompilerParams(needs_layout_passes=False),
        scratch_types=[
            pltpu.VMEM_SHARED((N * HALF_D,), jnp.float32),
            pltpu.VMEM((CHUNK, D), jnp.float32),
            pltpu.VMEM((UPD,), jnp.float32),
            pltpu.VMEM((UPD,), jnp.int32),
            pltpu.VMEM((1, CHUNK), jnp.int32),
            pltpu.VMEM((1, CHUNK), jnp.int32),
            pltpu.VMEM((1, CHUNK), jnp.float32),
        ],
    )(review3d, ci_flat, src3, dst3)


ROW_BLK = 1000


def _tc_body(h0_ref, h1_ref, w0_ref, w1_ref, ci_ref, o_ref):
    dn = (((1,), (1,)), ((), ()))
    hw = jax.lax.dot_general(h0_ref[...], w0_ref[...], dn,
                             preferred_element_type=jnp.float32)
    hw += jax.lax.dot_general(h1_ref[...], w1_ref[...], dn,
                              preferred_element_type=jnp.float32)
    o_ref[...] = hw * ci_ref[...]


def _tc_finish(h0, h1, W, ci):
    grid = (N // ROW_BLK,)
    W0 = W[:, :HALF_D]
    W1 = W[:, HALF_D:]
    return pl.pallas_call(
        _tc_body,
        grid=grid,
        in_specs=[
            pl.BlockSpec((ROW_BLK, HALF_D), lambda i: (i, 0)),
            pl.BlockSpec((ROW_BLK, HALF_D), lambda i: (i, 0)),
            pl.BlockSpec((D, HALF_D), lambda i: (0, 0)),
            pl.BlockSpec((D, HALF_D), lambda i: (0, 0)),
            pl.BlockSpec((ROW_BLK, 1), lambda i: (i, 0)),
        ],
        out_specs=pl.BlockSpec((ROW_BLK, D), lambda i: (i, 0)),
        out_shape=jax.ShapeDtypeStruct((N, D), jnp.float32),
    )(h0, h1, W0, W1, ci)


@jax.jit
def kernel(review_feat, ci, edge_index, W):
    review3d = review_feat.reshape(NCHUNK, CHUNK, D)
    src3 = edge_index[0].reshape(NCHUNK, 1, CHUNK)
    dst3 = edge_index[1].reshape(NCHUNK, 1, CHUNK)
    ci_flat = ci.reshape(N)
    h3 = _sc_aggregate(review3d, ci_flat, src3, dst3)
    h0 = h3[:N * HALF_D].reshape(N, HALF_D)
    h1 = h3[N * HALF_D:].reshape(N, HALF_D)
    return _tc_finish(h0, h1, W, ci)


# speedup vs baseline: 1.9987x; 1.2687x over previous
import functools

import jax
import jax.numpy as jnp
from jax import lax
from jax.experimental import pallas as pl
from jax.experimental.pallas import tpu as pltpu
from jax.experimental.pallas import tpu_sc as plsc

N = 50000
E = 800000
D = 64

CHUNK = 80
NS = 16
NC = 2
NCHUNK = E // CHUNK
WINDOWS = NCHUNK // NS
HALF_D = D // NC
ROWS_PER_TILE = N // NS
ZROWS = 125
ZFLAT = ZROWS * HALF_D
WB_ROWS = 3128
WB_LAST = N - (NS - 1) * WB_ROWS
UPD = CHUNK * HALF_D
CI_PAD = 51200


def _sc_body(review3d, ci_pad, src3, dst3, h_out,
             acc1d, ci_spmem,
             rows0, rows1, upd0, upd1, idx0, idx1,
             src0, src1, dst0, dst1, civ0, civ1,
             in_sem0, in_sem1, sc_sem0, sc_sem1):
    c = lax.axis_index("c")
    s = lax.axis_index("s")
    col0 = c * HALF_D
    rows_b = (rows0, rows1)
    upd_b = (upd0, upd1)
    idx_b = (idx0, idx1)
    src_b = (src0, src1)
    dst_b = (dst0, dst1)
    civ_b = (civ0, civ1)
    in_sem = (in_sem0, in_sem1)
    sc_sem = (sc_sem0, sc_sem1)

    @pl.when(s < 9)
    def _():
        for q in range(2):
            off = (2 * s + q) * UPD
            pltpu.sync_copy(ci_pad.at[pl.ds(off, UPD)], upd0)
            pltpu.sync_copy(upd0, ci_spmem.at[pl.ds(off, UPD)])

    @pl.when(s == 9)
    def _():
        off = 18 * UPD
        pltpu.sync_copy(ci_pad.at[pl.ds(off, UPD)], upd0)
        pltpu.sync_copy(upd0, ci_spmem.at[pl.ds(off, UPD)])
        off = 19 * UPD
        tail = N - off
        pltpu.sync_copy(ci_pad.at[pl.ds(off, UPD)], upd1)
        pltpu.sync_copy(upd1.at[pl.ds(0, tail)],
                        ci_spmem.at[pl.ds(off, tail)])

    def zstore(i, carry):
        upd0[pl.ds(i * 16, 16)] = jnp.zeros((16,), jnp.float32)
        return carry
    lax.fori_loop(0, UPD // 16, zstore, 0)
    NQ = (N * HALF_D) // UPD
    QPT = NQ // NS
    for k in range(QPT):
        pltpu.sync_copy(upd0, acc1d.at[pl.ds((s * QPT + k) * UPD, UPD)])

    @pl.when(s == NS - 1)
    def _():
        pltpu.sync_copy(upd0, acc1d.at[pl.ds((NQ - 1) * UPD, UPD)])

    plsc.subcore_barrier()

    iota16 = lax.iota(jnp.int32, 16)
    col0v = jnp.full((16,), 1, jnp.int32) * col0
    wbase = s * WINDOWS

    def issue_in(b, cb):
        pltpu.async_copy(review3d.at[cb], rows_b[b], in_sem[b])
        pltpu.async_copy(src3.at[cb], src_b[b], in_sem[b])
        pltpu.async_copy(dst3.at[cb], dst_b[b], in_sem[b])

    def wait_in(b, cb):
        pltpu.make_async_copy(review3d.at[cb], rows_b[b], in_sem[b]).wait()
        pltpu.make_async_copy(src3.at[cb], src_b[b], in_sem[b]).wait()
        pltpu.make_async_copy(dst3.at[cb], dst_b[b], in_sem[b]).wait()

    def compute(b):
        rows, upd, idx = rows_b[b], upd_b[b], idx_b[b]
        for g in range(CHUNK // 16):
            sl = pl.ds(g * 16, 16)
            civ = civ_b[b][0, sl]
            dstv = dst_b[b][0, sl]
            dst32 = dstv * HALF_D
            eidx = g * 16 + iota16
            pos0 = eidx * HALF_D

            rpos0 = eidx * D + col0v

            def dbody(d, carry2, civ=civ, dst32=dst32, pos0=pos0,
                      rows=rows, upd=upd, idx=idx, rpos0=rpos0):
                dvec = jnp.full((16,), 1, jnp.int32) * d
                col = plsc.load_gather(rows, [jnp.zeros((16,), jnp.int32),
                                              rpos0 + dvec])
                plsc.store_scatter(upd, [pos0 + dvec], col * civ)
                plsc.store_scatter(idx, [pos0 + dvec], dst32 + dvec)
                return carry2
            lax.fori_loop(0, HALF_D, dbody, 0, unroll=4)

    def do_window(b, wb, cb):
        issue_in(1 - b, cb + 1)
        wait_in(b, cb)
        pltpu.sync_copy(ci_spmem.at[src_b[b].at[0]], civ_b[b].at[0])

        @pl.when(wb >= 2)
        def _():
            pltpu.make_async_copy(upd_b[b], acc1d.at[idx_b[b]],
                                  sc_sem[b]).wait()
        compute(b)
        pltpu.async_copy(upd_b[b], acc1d.at[idx_b[b]], sc_sem[b], add=True)

    issue_in(0, wbase)

    def pipe(i, carry):
        for b in range(2):
            wb = 2 * i + b
            do_window(b, wb, wbase + wb)
        return carry
    lax.fori_loop(0, (WINDOWS - 1) // 2, pipe, 0)

    wb = WINDOWS - 1
    cb = wbase + wb
    wait_in(0, cb)
    pltpu.sync_copy(ci_spmem.at[src_b[0].at[0]], civ_b[0].at[0])
    pltpu.make_async_copy(upd_b[0], acc1d.at[idx_b[0]], sc_sem[0]).wait()
    compute(0)
    pltpu.async_copy(upd_b[0], acc1d.at[idx_b[0]], sc_sem[0], add=True)
    pltpu.make_async_copy(upd_b[1], acc1d.at[idx_b[1]], sc_sem[1]).wait()
    pltpu.make_async_copy(upd_b[0], acc1d.at[idx_b[0]], sc_sem[0]).wait()

    plsc.subcore_barrier()

    hbase = c * (N * HALF_D)
    for k in range(QPT):
        off = (s * QPT + k) * UPD
        pltpu.sync_copy(acc1d.at[pl.ds(off, UPD)], upd0)
        pltpu.sync_copy(upd0, h_out.at[pl.ds(hbase + off, UPD)])

    @pl.when(s == NS - 1)
    def _():
        off = (NQ - 1) * UPD
        pltpu.sync_copy(acc1d.at[pl.ds(off, UPD)], upd0)
        pltpu.sync_copy(upd0, h_out.at[pl.ds(hbase + off, UPD)])


def _sc_aggregate(review3d, ci_pad, src3, dst3):
    mesh = plsc.VectorSubcoreMesh(core_axis_name="c", subcore_axis_name="s")
    return pl.kernel(
        _sc_body,
        out_type=jax.ShapeDtypeStruct((NC * N * HALF_D,), jnp.float32),
        mesh=mesh,
        compiler_params=pltpu.CompilerParams(needs_layout_passes=False),
        scratch_types=[
            pltpu.VMEM_SHARED((N * HALF_D,), jnp.float32),
            pltpu.VMEM_SHARED((N,), jnp.float32),
            pltpu.VMEM((1, CHUNK * D), jnp.float32),
            pltpu.VMEM((1, CHUNK * D), jnp.float32),
            pltpu.VMEM((UPD,), jnp.float32),
            pltpu.VMEM((UPD,), jnp.float32),
            pltpu.VMEM((UPD,), jnp.int32),
            pltpu.VMEM((UPD,), jnp.int32),
            pltpu.VMEM((1, CHUNK), jnp.int32),
            pltpu.VMEM((1, CHUNK), jnp.int32),
            pltpu.VMEM((1, CHUNK), jnp.int32),
            pltpu.VMEM((1, CHUNK), jnp.int32),
            pltpu.VMEM((1, CHUNK), jnp.float32),
            pltpu.VMEM((1, CHUNK), jnp.float32),
            pltpu.SemaphoreType.DMA,
            pltpu.SemaphoreType.DMA,
            pltpu.SemaphoreType.DMA,
            pltpu.SemaphoreType.DMA,
        ],
    )(review3d, ci_pad, src3, dst3)


ROW_BLK = 1000


def _tc_body(h0_ref, h1_ref, w0_ref, w1_ref, ci_ref, o_ref):
    dn = (((1,), (1,)), ((), ()))
    hw = jax.lax.dot_general(h0_ref[...], w0_ref[...], dn,
                             preferred_element_type=jnp.float32)
    hw += jax.lax.dot_general(h1_ref[...], w1_ref[...], dn,
                              preferred_element_type=jnp.float32)
    o_ref[...] = hw * ci_ref[...]


def _tc_finish(h0, h1, W, ci):
    grid = (N // ROW_BLK,)
    W0 = W[:, :HALF_D]
    W1 = W[:, HALF_D:]
    return pl.pallas_call(
        _tc_body,
        grid=grid,
        in_specs=[
            pl.BlockSpec((ROW_BLK, HALF_D), lambda i: (i, 0)),
            pl.BlockSpec((ROW_BLK, HALF_D), lambda i: (i, 0)),
            pl.BlockSpec((D, HALF_D), lambda i: (0, 0)),
            pl.BlockSpec((D, HALF_D), lambda i: (0, 0)),
            pl.BlockSpec((ROW_BLK, 1), lambda i: (i, 0)),
        ],
        out_specs=pl.BlockSpec((ROW_BLK, D), lambda i: (i, 0)),
        out_shape=jax.ShapeDtypeStruct((N, D), jnp.float32),
    )(h0, h1, W0, W1, ci)


@jax.jit
def kernel(review_feat, ci, edge_index, W):
    review3d = review_feat.reshape(NCHUNK, 1, CHUNK * D)
    src3 = edge_index[0].reshape(NCHUNK, 1, CHUNK)
    dst3 = edge_index[1].reshape(NCHUNK, 1, CHUNK)
    ci_pad = jnp.pad(ci.reshape(N), (0, CI_PAD - N))
    h3 = _sc_aggregate(review3d, ci_pad, src3, dst3)
    h0 = h3[:N * HALF_D].reshape(N, HALF_D)
    h1 = h3[N * HALF_D:].reshape(N, HALF_D)
    return _tc_finish(h0, h1, W, ci)


# speedup vs baseline: 4.6937x; 2.3483x over previous
import functools

import jax
import jax.numpy as jnp
from jax import lax
from jax.experimental import pallas as pl
from jax.experimental.pallas import tpu as pltpu
from jax.experimental.pallas import tpu_sc as plsc

N = 50000
E = 800000
D = 64

CHUNK = 40
NS = 16
NC = 2
NCHUNK = E // CHUNK
WINDOWS = NCHUNK // NS
HALF_D = D // NC
ROWS_PER_TILE = N // NS
ZROWS = 125
ZFLAT = ZROWS * HALF_D
WB_ROWS = 3128
WB_LAST = N - (NS - 1) * WB_ROWS
UPD = CHUNK * HALF_D
CI_PAD = 51200


def _sc_body(review2d, ci_pad, src3, dst3, h_out,
             acc1d,
             rows0, rows1, upd0, upd1, idx0, idx1,
             src0, src1, dst0, dst1, civ0,
             in_sem0, in_sem1, sc_sem0, sc_sem1, ci_sem0, ci_sem1):
    c = lax.axis_index("c")
    s = lax.axis_index("s")
    col0 = c * HALF_D
    rows_b = (rows0, rows1)
    upd_b = (upd0, upd1)
    idx_b = (idx0, idx1)
    src_b = (src0, src1)
    dst_b = (dst0, dst1)
    civ_b = (civ0, civ0)
    in_sem = (in_sem0, in_sem1)
    sc_sem = (sc_sem0, sc_sem1)
    ci_sem = (ci_sem0, ci_sem1)

    def zstore(i, carry):
        upd0[pl.ds(i * 16, 16)] = jnp.zeros((16,), jnp.float32)
        return carry
    lax.fori_loop(0, UPD // 16, zstore, 0)
    NQ = (N * HALF_D) // UPD
    QPT = NQ // NS
    QREM = NQ - QPT * NS
    for k in range(QPT):
        pltpu.sync_copy(upd0, acc1d.at[pl.ds((s * QPT + k) * UPD, UPD)])

    @pl.when(s >= NS - QREM)
    def _():
        pltpu.sync_copy(upd0, acc1d.at[pl.ds((NQ - NS + s) * UPD, UPD)])

    plsc.subcore_barrier()

    iota16 = lax.iota(jnp.int32, 16)
    col0v = jnp.full((16,), 1, jnp.int32) * col0
    wbase = s * WINDOWS

    def issue_in(b, cb):
        pltpu.async_copy(review2d.at[pl.ds(cb * CHUNK, CHUNK)], rows_b[b],
                         in_sem[b])
        pltpu.async_copy(src3.at[cb], src_b[b], in_sem[b])
        pltpu.async_copy(dst3.at[cb], dst_b[b], in_sem[b])

    def wait_in(b, cb):
        pltpu.make_async_copy(review2d.at[pl.ds(cb * CHUNK, CHUNK)],
                              rows_b[b], in_sem[b]).wait()
        pltpu.make_async_copy(src3.at[cb], src_b[b], in_sem[b]).wait()
        pltpu.make_async_copy(dst3.at[cb], dst_b[b], in_sem[b]).wait()

    zero16 = jnp.zeros((16,), jnp.int32)

    one16 = jnp.full((16,), 1, jnp.int32)

    def compute_idx(b):
        idx = idx_b[b]

        def ebody(e, carry):
            efull = one16 * e
            dsp = plsc.load_gather(dst_b[b], [zero16, efull])
            dsp32 = dsp * HALF_D
            for q in range(HALF_D // 16):
                idx[pl.ds(e * HALF_D + q * 16, 16)] = dsp32 + (iota16 + q * 16)
            return carry
        lax.fori_loop(0, CHUNK, ebody, 0, unroll=8)

    def compute_upd(b):
        rows, upd = rows_b[b], upd_b[b]

        def ebody(e, carry):
            efull = one16 * e
            csp = plsc.load_gather(civ_b[b], [zero16, efull])
            for q in range(HALF_D // 16):
                v = rows[e, pl.ds(col0 + q * 16, 16)]
                upd[pl.ds(e * HALF_D + q * 16, 16)] = v * csp
            return carry
        lax.fori_loop(0, CHUNK, ebody, 0, unroll=8)

    def do_window(b, wb, cb, prefetch=True, first=False):
        if prefetch:
            issue_in(1 - b, cb + 1)
        wait_in(b, cb)
        pltpu.async_copy(ci_pad.at[src_b[b].at[0]], civ_b[b].at[0], ci_sem[b])

        if first:
            @pl.when(wb >= 2)
            def _():
                pltpu.make_async_copy(upd_b[b], acc1d.at[idx_b[b]],
                                      sc_sem[b]).wait()
        else:
            pltpu.make_async_copy(upd_b[b], acc1d.at[idx_b[b]],
                                  sc_sem[b]).wait()
        compute_idx(b)
        pltpu.make_async_copy(ci_pad.at[src_b[b].at[0]], civ_b[b].at[0],
                              ci_sem[b]).wait()
        compute_upd(b)
        pltpu.async_copy(upd_b[b], acc1d.at[idx_b[b]], sc_sem[b], add=True)

    issue_in(0, wbase)

    def pipe(i, carry):
        for b in range(2):
            wb = 2 * i + b
            do_window(b, wb, wbase + wb, first=True)
        return carry
    lax.fori_loop(0, (WINDOWS - 2) // 2, pipe, 0)

    do_window(0, WINDOWS - 2, wbase + WINDOWS - 2, prefetch=True)
    do_window(1, WINDOWS - 1, wbase + WINDOWS - 1, prefetch=False)
    pltpu.make_async_copy(upd_b[0], acc1d.at[idx_b[0]], sc_sem[0]).wait()
    pltpu.make_async_copy(upd_b[1], acc1d.at[idx_b[1]], sc_sem[1]).wait()

    plsc.subcore_barrier()

    hbase = c * (N * HALF_D)
    for k in range(QPT):
        off = (s * QPT + k) * UPD
        pltpu.sync_copy(acc1d.at[pl.ds(off, UPD)], upd0)
        pltpu.sync_copy(upd0, h_out.at[pl.ds(hbase + off, UPD)])

    @pl.when(s >= NS - QREM)
    def _():
        off = (NQ - NS + s) * UPD
        pltpu.sync_copy(acc1d.at[pl.ds(off, UPD)], upd0)
        pltpu.sync_copy(upd0, h_out.at[pl.ds(hbase + off, UPD)])


def _sc_aggregate(review2d, ci_pad, src3, dst3):
    mesh = plsc.VectorSubcoreMesh(core_axis_name="c", subcore_axis_name="s")
    return pl.kernel(
        _sc_body,
        out_type=jax.ShapeDtypeStruct((NC * N * HALF_D,), jnp.float32),
        mesh=mesh,
        compiler_params=pltpu.CompilerParams(needs_layout_passes=False),
        scratch_types=[
            pltpu.VMEM_SHARED((N * HALF_D,), jnp.float32),
            pltpu.VMEM((CHUNK, D), jnp.float32),
            pltpu.VMEM((CHUNK, D), jnp.float32),
            pltpu.VMEM((UPD,), jnp.float32),
            pltpu.VMEM((UPD,), jnp.float32),
            pltpu.VMEM((UPD,), jnp.int32),
            pltpu.VMEM((UPD,), jnp.int32),
            pltpu.VMEM((1, CHUNK), jnp.int32),
            pltpu.VMEM((1, CHUNK), jnp.int32),
            pltpu.VMEM((1, CHUNK), jnp.int32),
            pltpu.VMEM((1, CHUNK), jnp.int32),
            pltpu.VMEM((1, CHUNK), jnp.float32),
            pltpu.SemaphoreType.DMA,
            pltpu.SemaphoreType.DMA,
            pltpu.SemaphoreType.DMA,
            pltpu.SemaphoreType.DMA,
            pltpu.SemaphoreType.DMA,
            pltpu.SemaphoreType.DMA,
        ],
    )(review2d, ci_pad, src3, dst3)


ROW_BLK = 1000


def _tc_body(h0_ref, h1_ref, w0_ref, w1_ref, ci_ref, o_ref):
    dn = (((1,), (1,)), ((), ()))
    hw = jax.lax.dot_general(h0_ref[...], w0_ref[...], dn,
                             preferred_element_type=jnp.float32)
    hw += jax.lax.dot_general(h1_ref[...], w1_ref[...], dn,
                              preferred_element_type=jnp.float32)
    o_ref[...] = hw * ci_ref[...]


def _tc_finish(h0, h1, W, ci):
    grid = (N // ROW_BLK,)
    W0 = W[:, :HALF_D]
    W1 = W[:, HALF_D:]
    return pl.pallas_call(
        _tc_body,
        grid=grid,
        in_specs=[
            pl.BlockSpec((ROW_BLK, HALF_D), lambda i: (i, 0)),
            pl.BlockSpec((ROW_BLK, HALF_D), lambda i: (i, 0)),
            pl.BlockSpec((D, HALF_D), lambda i: (0, 0)),
            pl.BlockSpec((D, HALF_D), lambda i: (0, 0)),
            pl.BlockSpec((ROW_BLK, 1), lambda i: (i, 0)),
        ],
        out_specs=pl.BlockSpec((ROW_BLK, D), lambda i: (i, 0)),
        out_shape=jax.ShapeDtypeStruct((N, D), jnp.float32),
    )(h0, h1, W0, W1, ci)


@jax.jit
def kernel(review_feat, ci, edge_index, W):
    src3 = edge_index[0].reshape(NCHUNK, 1, CHUNK)
    dst3 = edge_index[1].reshape(NCHUNK, 1, CHUNK)
    ci_pad = jnp.pad(ci.reshape(N), (0, CI_PAD - N))
    h3 = _sc_aggregate(review_feat, ci_pad, src3, dst3)
    h0 = h3[:N * HALF_D].reshape(N, HALF_D)
    h1 = h3[N * HALF_D:].reshape(N, HALF_D)
    return _tc_finish(h0, h1, W, ci)
